# Initial kernel scaffold; baseline (speedup 1.0000x reference)
import functools

import jax
import jax.numpy as jnp
from jax import lax
from jax.experimental import pallas as pl
from jax.experimental.pallas import tpu as pltpu
from jax.experimental.pallas import tpu_sc as plsc

N = 10000
E = 320000
D = 128
H = 128

NC = 2
NS = 16
NW = NC * NS
ROWS = D // NW

CH = 3200
NCHUNK = E // CH
GRP = CH // 16
CNT_PER = E // NW
CCH = 2000
NCCH = CNT_PER // CCH


def _sc_body(with_counts, y_hbm, edges_hbm, s_out, *rest):
    if with_counts:
        cnt_out, table_v, acc_v, src_v, dst_v, cnt_v = rest
    else:
        table_v, acc_v, src_v, dst_v = rest

    wid = lax.axis_index("s") * NC + lax.axis_index("c")

    z16f = jnp.zeros((16,), jnp.float32)
    ones16 = jnp.ones((16,), jnp.float32)
    z16i = jnp.zeros((16,), jnp.int32)

    pltpu.sync_copy(y_hbm.at[pl.ds(ROWS * wid, ROWS)], table_v)

    for c in range(ROWS):
        @pl.loop(0, N // 16, unroll=8)
        def _(i, c=c):
            acc_v[c, pl.ds(i * 16, 16)] = z16f

    if with_counts:
        @pl.loop(0, N // 16, unroll=8)
        def _(i):
            cnt_v[0, pl.ds(i * 16, 16)] = z16f

        @pl.loop(0, NCCH)
        def _(k):
            pltpu.sync_copy(
                edges_hbm.at[1, pl.ds(wid * CNT_PER + k * CCH, CCH)],
                dst_v.at[pl.ds(0, CCH)],
            )

            @pl.loop(0, CCH // 16, unroll=8)
            def _(g):
                d16 = dst_v[pl.ds(g * 16, 16)]
                plsc.addupdate_scatter(cnt_v, [z16i, d16], ones16)

        pltpu.sync_copy(cnt_v, cnt_out.at[pl.ds(wid, 1)])

    @pl.loop(0, NCHUNK)
    def _(k):
        pltpu.sync_copy(edges_hbm.at[0, pl.ds(k * CH, CH)], src_v)
        pltpu.sync_copy(edges_hbm.at[1, pl.ds(k * CH, CH)], dst_v)

        @pl.loop(0, GRP, unroll=8)
        def _(g):
            s16 = src_v[pl.ds(g * 16, 16)]
            d16 = dst_v[pl.ds(g * 16, 16)]
            for c in range(ROWS):
                cv = jnp.full((16,), c, jnp.int32)
                v = plsc.load_gather(table_v, [cv, s16])
                plsc.addupdate_scatter(acc_v, [cv, d16], v)

    pltpu.sync_copy(acc_v, s_out.at[pl.ds(ROWS * wid, ROWS)])


def _make_sc_kernel(with_counts):
    outs = [jax.ShapeDtypeStruct((D, N), jnp.float32)]
    scratch = [
        pltpu.VMEM((ROWS, N), jnp.float32),
        pltpu.VMEM((ROWS, N), jnp.float32),
        pltpu.VMEM((CH,), jnp.int32),
        pltpu.VMEM((CH,), jnp.int32),
    ]
    if with_counts:
        outs.append(jax.ShapeDtypeStruct((NW, N), jnp.float32))
        scratch.append(pltpu.VMEM((1, N), jnp.float32))
    mesh = plsc.VectorSubcoreMesh(core_axis_name="c", subcore_axis_name="s")
    return pl.kernel(
        functools.partial(_sc_body, with_counts),
        out_type=tuple(outs) if with_counts else outs[0],
        mesh=mesh,
        scratch_types=scratch,
        name="sage_seg_sum" + ("_cnt" if with_counts else ""),
    )


_sc_sum_cnt = _make_sc_kernel(True)
_sc_sum = _make_sc_kernel(False)


BN = 1000


def _tc_pre_body(xt_ref, wl1_ref, y1t_ref):
    y1t_ref[...] = lax.dot_general(
        wl1_ref[...], xt_ref[...], (((0,), (0,)), ((), ())),
        preferred_element_type=jnp.float32)


def _tc_mid_body(s1_ref, cnt_ref, xt_ref, wr1_ref, wl2_ref, wr2_ref, b1_ref,
                 b2_ref, y2t_ref, hr2t_ref):
    c = jnp.sum(cnt_ref[...], axis=0, keepdims=True)
    cmax = jnp.maximum(c, 1.0)
    m1t = s1_ref[...] / cmax
    xr = lax.dot_general(wr1_ref[...], xt_ref[...], (((0,), (0,)), ((), ())),
                         preferred_element_type=jnp.float32)
    h = jnp.maximum(m1t + xr + b1_ref[...], 0.0)
    y2t_ref[...] = lax.dot_general(wl2_ref[...], h, (((0,), (0,)), ((), ())),
                                   preferred_element_type=jnp.float32)
    hr2t_ref[...] = lax.dot_general(wr2_ref[...], h, (((0,), (0,)), ((), ())),
                                    preferred_element_type=jnp.float32) + b2_ref[...]


def _tc_dec_body(s2_ref, cnt_ref, hr2_ref, dw1_ref, db1_ref, dw2t_ref,
                 db2_ref, out_ref):
    c = jnp.sum(cnt_ref[...], axis=0, keepdims=True)
    cmax = jnp.maximum(c, 1.0)
    h2 = s2_ref[...] / cmax + hr2_ref[...]
    z = jnp.maximum(
        lax.dot_general(dw1_ref[...], h2, (((0,), (0,)), ((), ())),
                        preferred_element_type=jnp.float32) + db1_ref[...], 0.0)
    out_ref[...] = lax.dot_general(dw2t_ref[...], z, (((1,), (0,)), ((), ())),
                                   preferred_element_type=jnp.float32) + db2_ref[...]


def _full(shape):
    return pl.BlockSpec(shape, lambda i: (0, 0))


def _blk(rows):
    return pl.BlockSpec((rows, BN), lambda i: (0, i))


_tc_pre = pl.pallas_call(
    _tc_pre_body,
    grid=(N // BN,),
    in_specs=[_blk(D), _full((D, H))],
    out_specs=_blk(H),
    out_shape=jax.ShapeDtypeStruct((H, N), jnp.float32),
)

_tc_mid = pl.pallas_call(
    _tc_mid_body,
    grid=(N // BN,),
    in_specs=[_blk(H), _blk(NW), _blk(D), _full((D, H)), _full((H, H)),
              _full((H, H)), _full((H, 1)), _full((H, 1))],
    out_specs=[_blk(H), _blk(H)],
    out_shape=[jax.ShapeDtypeStruct((H, N), jnp.float32),
               jax.ShapeDtypeStruct((H, N), jnp.float32)],
)

_tc_dec = pl.pallas_call(
    _tc_dec_body,
    grid=(N // BN,),
    in_specs=[_blk(H), _blk(NW), _blk(H), _full((H, H)), _full((H, 1)),
              _full((1, H)), _full((1, 1))],
    out_specs=_blk(1),
    out_shape=jax.ShapeDtypeStruct((1, N), jnp.float32),
)


def kernel(x, edge_index, W_l1, W_r1, b1, W_l2, W_r2, b2, dec_w1, dec_b1,
           dec_w2, dec_b2):
    xt = x.T
    y1t = _tc_pre(xt, W_l1)
    s1t, cnt = _sc_sum_cnt(y1t, edge_index)
    y2t, hr2t = _tc_mid(s1t, cnt, xt, W_r1, W_l2, W_r2,
                        b1.reshape(H, 1), b2.reshape(H, 1))
    s2t = _sc_sum(y2t, edge_index)
    out = _tc_dec(s2t, cnt, hr2t, dec_w1, dec_b1.reshape(H, 1),
                  dec_w2.T, dec_b2.reshape(1, 1))
    return out.reshape(-1)


# speedup vs baseline: 2.7675x; 2.7675x over previous
import functools

import jax
import jax.numpy as jnp
from jax import lax
from jax.experimental import pallas as pl
from jax.experimental.pallas import tpu as pltpu
from jax.experimental.pallas import tpu_sc as plsc

N = 10000
E = 320000
D = 128
H = 128

NC = 2
NS = 16
NW = NC * NS
ROWS = D // NW

CH = 3200
NCHUNK = E // CH
GRP = CH // 16
CNT_PER = E // NW
CCH = 2000
NCCH = CNT_PER // CCH


def _sc_body(with_counts, y_hbm, src_hbm, dst_hbm, s_out, *rest):
    if with_counts:
        cnt_out = rest[0]
        rest = rest[1:]
    table_v = rest[0:ROWS]
    acc_v = rest[ROWS:2 * ROWS]
    src_v = rest[2 * ROWS]
    dst_v = rest[2 * ROWS + 1]
    if with_counts:
        cnt_v = rest[2 * ROWS + 2]

    wid = lax.axis_index("s") * NC + lax.axis_index("c")

    z16f = jnp.zeros((16,), jnp.float32)
    ones16 = jnp.ones((16,), jnp.float32)

    for c in range(ROWS):
        pltpu.sync_copy(y_hbm.at[pl.ds((ROWS * wid + c) * N, N)], table_v[c])

    for c in range(ROWS):
        @pl.loop(0, N // 16, unroll=8)
        def _(i, c=c):
            acc_v[c][pl.ds(i * 16, 16)] = z16f

    if with_counts:
        @pl.loop(0, N // 16, unroll=8)
        def _(i):
            cnt_v[pl.ds(i * 16, 16)] = z16f

        @pl.loop(0, NCCH)
        def _(k):
            pltpu.sync_copy(
                dst_hbm.at[pl.ds(wid * CNT_PER + k * CCH, CCH)],
                dst_v.at[pl.ds(0, CCH)],
            )

            @pl.loop(0, CCH // 16, unroll=8)
            def _(g):
                d16 = dst_v[pl.ds(g * 16, 16)]
                plsc.addupdate_scatter(cnt_v, [d16], ones16)

        pltpu.sync_copy(cnt_v, cnt_out.at[pl.ds(wid * N, N)])

    @pl.loop(0, NCHUNK)
    def _(k):
        pltpu.sync_copy(src_hbm.at[pl.ds(k * CH, CH)], src_v)
        pltpu.sync_copy(dst_hbm.at[pl.ds(k * CH, CH)], dst_v)

        @pl.loop(0, GRP, unroll=8)
        def _(g):
            s16 = src_v[pl.ds(g * 16, 16)]
            d16 = dst_v[pl.ds(g * 16, 16)]
            for c in range(ROWS):
                v = plsc.load_gather(table_v[c], [s16])
                plsc.addupdate_scatter(acc_v[c], [d16], v)

    for c in range(ROWS):
        pltpu.sync_copy(acc_v[c], s_out.at[pl.ds((ROWS * wid + c) * N, N)])


def _make_sc_kernel(with_counts):
    outs = [jax.ShapeDtypeStruct((D * N,), jnp.float32)]
    scratch = (
        [pltpu.VMEM((N,), jnp.float32) for _ in range(ROWS)]
        + [pltpu.VMEM((N,), jnp.float32) for _ in range(ROWS)]
        + [pltpu.VMEM((CH,), jnp.int32),
           pltpu.VMEM((CH,), jnp.int32)]
    )
    if with_counts:
        outs.append(jax.ShapeDtypeStruct((NW * N,), jnp.float32))
        scratch.append(pltpu.VMEM((N,), jnp.float32))
    mesh = plsc.VectorSubcoreMesh(core_axis_name="c", subcore_axis_name="s")
    return pl.kernel(
        functools.partial(_sc_body, with_counts),
        out_type=tuple(outs) if with_counts else outs[0],
        mesh=mesh,
        scratch_types=scratch,
        compiler_params=pltpu.CompilerParams(needs_layout_passes=False),
        name="sage_seg_sum" + ("_cnt" if with_counts else ""),
    )


_sc_sum_cnt = _make_sc_kernel(True)
_sc_sum = _make_sc_kernel(False)


def _tc_pre_body(xt_ref, wl1_ref, y1t_ref):
    y1t_ref[...] = lax.dot_general(
        wl1_ref[...], xt_ref[...], (((0,), (0,)), ((), ())),
        preferred_element_type=jnp.float32)


def _tc_mid_body(s1_ref, cnt_ref, xt_ref, wr1_ref, wl2_ref, wr2_ref, b1_ref,
                 b2_ref, y2t_ref, hr2t_ref):
    c = jnp.sum(cnt_ref[...], axis=0, keepdims=True)
    cmax = jnp.maximum(c, 1.0)
    m1t = s1_ref[...] / cmax
    xr = lax.dot_general(wr1_ref[...], xt_ref[...], (((0,), (0,)), ((), ())),
                         preferred_element_type=jnp.float32)
    h = jnp.maximum(m1t + xr + b1_ref[...], 0.0)
    y2t_ref[...] = lax.dot_general(wl2_ref[...], h, (((0,), (0,)), ((), ())),
                                   preferred_element_type=jnp.float32)
    hr2t_ref[...] = lax.dot_general(wr2_ref[...], h, (((0,), (0,)), ((), ())),
                                    preferred_element_type=jnp.float32) + b2_ref[...]


def _tc_dec_body(s2_ref, cnt_ref, hr2_ref, dw1_ref, db1_ref, dw2t_ref,
                 db2_ref, out_ref):
    c = jnp.sum(cnt_ref[...], axis=0, keepdims=True)
    cmax = jnp.maximum(c, 1.0)
    h2 = s2_ref[...] / cmax + hr2_ref[...]
    z = jnp.maximum(
        lax.dot_general(dw1_ref[...], h2, (((0,), (0,)), ((), ())),
                        preferred_element_type=jnp.float32) + db1_ref[...], 0.0)
    out_ref[...] = lax.dot_general(dw2t_ref[...], z, (((1,), (0,)), ((), ())),
                                   preferred_element_type=jnp.float32) + db2_ref[...]


_tc_pre = pl.pallas_call(
    _tc_pre_body,
    out_shape=jax.ShapeDtypeStruct((H, N), jnp.float32),
)

_tc_mid = pl.pallas_call(
    _tc_mid_body,
    out_shape=[jax.ShapeDtypeStruct((H, N), jnp.float32),
               jax.ShapeDtypeStruct((H, N), jnp.float32)],
)

_tc_dec = pl.pallas_call(
    _tc_dec_body,
    out_shape=jax.ShapeDtypeStruct((1, N), jnp.float32),
)


def kernel(x, edge_index, W_l1, W_r1, b1, W_l2, W_r2, b2, dec_w1, dec_b1,
           dec_w2, dec_b2):
    xt = x.T
    src = edge_index[0]
    dst = edge_index[1]
    y1t = _tc_pre(xt, W_l1)
    s1t, cnt = _sc_sum_cnt(y1t.reshape(-1), src, dst)
    s1t = s1t.reshape(H, N)
    cnt = cnt.reshape(NW, N)
    y2t, hr2t = _tc_mid(s1t, cnt, xt, W_r1, W_l2, W_r2,
                        b1.reshape(H, 1), b2.reshape(H, 1))
    s2t = _sc_sum(y2t.reshape(-1), src, dst).reshape(H, N)
    out = _tc_dec(s2t, cnt, hr2t, dec_w1, dec_b1.reshape(H, 1),
                  dec_w2.T, dec_b2.reshape(1, 1))
    return out.reshape(-1)


# speedup vs baseline: 3.5196x; 1.2718x over previous
import functools

import jax
import jax.numpy as jnp
from jax import lax
from jax.experimental import pallas as pl
from jax.experimental.pallas import tpu as pltpu
from jax.experimental.pallas import tpu_sc as plsc

N = 10000
E = 320000
D = 128
H = 128

NC = 2
NS = 16
NW = NC * NS
ROWS = D // NW

CH = 3200
NCHUNK = E // CH
GRP = CH // 16
CNT_PER = E // NW
CCH = 2000
NCCH = CNT_PER // CCH


def _sc_body(with_counts, y_hbm, src_hbm, dst_hbm, s_out, *rest):
    if with_counts:
        cnt_out = rest[0]
        rest = rest[1:]
    table_v = rest[0:ROWS]
    acc_v = rest[ROWS:2 * ROWS]
    src_b = rest[2 * ROWS:2 * ROWS + 2]
    dst_b = rest[2 * ROWS + 2:2 * ROWS + 4]
    sem_s = rest[2 * ROWS + 4]
    sem_d = rest[2 * ROWS + 5]
    if with_counts:
        cnt_v = rest[2 * ROWS + 6]

    wid = lax.axis_index("s") * NC + lax.axis_index("c")

    z16f = jnp.zeros((16,), jnp.float32)
    ones16 = jnp.ones((16,), jnp.float32)

    for c in range(ROWS):
        pltpu.async_copy(y_hbm.at[pl.ds((ROWS * wid + c) * N, N)],
                         table_v[c], sem_s)

    for b in range(2):
        pltpu.async_copy(src_hbm.at[pl.ds(b * CH, CH)], src_b[b], sem_s)
        pltpu.async_copy(dst_hbm.at[pl.ds(b * CH, CH)], dst_b[b], sem_d)

    for c in range(ROWS):
        @pl.loop(0, N // 16, unroll=8)
        def _(i, c=c):
            acc_v[c][pl.ds(i * 16, 16)] = z16f

    if with_counts:
        @pl.loop(0, N // 16, unroll=8)
        def _(i):
            cnt_v[pl.ds(i * 16, 16)] = z16f

    for c in range(ROWS):
        pltpu.make_async_copy(y_hbm.at[pl.ds(0, N)], table_v[c], sem_s).wait()

    @pl.loop(0, NCHUNK, step=2)
    def _(k):
        for b in range(2):
            sv, dv = src_b[b], dst_b[b]
            pltpu.make_async_copy(src_hbm.at[pl.ds(0, CH)], sv, sem_s).wait()
            pltpu.make_async_copy(dst_hbm.at[pl.ds(0, CH)], dv, sem_d).wait()

            @pl.loop(0, GRP, unroll=8)
            def _(g, sv=sv, dv=dv):
                s16 = sv[pl.ds(g * 16, 16)]
                d16 = dv[pl.ds(g * 16, 16)]
                for c in range(ROWS):
                    v = plsc.load_gather(table_v[c], [s16])
                    plsc.addupdate_scatter(acc_v[c], [d16], v)

            nxt = k + 2 + b

            @pl.when(nxt < NCHUNK)
            def _(sv=sv, dv=dv, nxt=nxt):
                pltpu.async_copy(src_hbm.at[pl.ds(nxt * CH, CH)], sv, sem_s)
                pltpu.async_copy(dst_hbm.at[pl.ds(nxt * CH, CH)], dv, sem_d)

    if with_counts:
        @pl.loop(0, NCCH)
        def _(k):
            pltpu.sync_copy(
                dst_hbm.at[pl.ds(wid * CNT_PER + k * CCH, CCH)],
                dst_b[0].at[pl.ds(0, CCH)],
            )

            @pl.loop(0, CCH // 16, unroll=8)
            def _(g):
                d16 = dst_b[0][pl.ds(g * 16, 16)]
                plsc.addupdate_scatter(cnt_v, [d16], ones16)

        pltpu.sync_copy(cnt_v, cnt_out.at[pl.ds(wid * N, N)])

    for c in range(ROWS):
        pltpu.sync_copy(acc_v[c], s_out.at[pl.ds((ROWS * wid + c) * N, N)])


def _make_sc_kernel(with_counts):
    outs = [jax.ShapeDtypeStruct((D * N,), jnp.float32)]
    scratch = (
        [pltpu.VMEM((N,), jnp.float32) for _ in range(ROWS)]
        + [pltpu.VMEM((N,), jnp.float32) for _ in range(ROWS)]
        + [pltpu.VMEM((CH,), jnp.int32) for _ in range(2)]
        + [pltpu.VMEM((CH,), jnp.int32) for _ in range(2)]
        + [pltpu.SemaphoreType.DMA, pltpu.SemaphoreType.DMA]
    )
    if with_counts:
        outs.append(jax.ShapeDtypeStruct((NW * N,), jnp.float32))
        scratch.append(pltpu.VMEM((N,), jnp.float32))
    mesh = plsc.VectorSubcoreMesh(core_axis_name="c", subcore_axis_name="s")
    return pl.kernel(
        functools.partial(_sc_body, with_counts),
        out_type=tuple(outs) if with_counts else outs[0],
        mesh=mesh,
        scratch_types=scratch,
        compiler_params=pltpu.CompilerParams(needs_layout_passes=False),
        name="sage_seg_sum" + ("_cnt" if with_counts else ""),
    )


_sc_sum_cnt = _make_sc_kernel(True)
_sc_sum = _make_sc_kernel(False)


def _tc_pre_body(xt_ref, wl1_ref, y1t_ref):
    y1t_ref[...] = lax.dot_general(
        wl1_ref[...], xt_ref[...], (((0,), (0,)), ((), ())),
        preferred_element_type=jnp.float32)


def _tc_mid_body(s1_ref, cnt_ref, xt_ref, wr1_ref, wl2_ref, wr2_ref, b1_ref,
                 b2_ref, y2t_ref, hr2t_ref):
    c = jnp.sum(cnt_ref[...], axis=0, keepdims=True)
    cmax = jnp.maximum(c, 1.0)
    m1t = s1_ref[...] / cmax
    xr = lax.dot_general(wr1_ref[...], xt_ref[...], (((0,), (0,)), ((), ())),
                         preferred_element_type=jnp.float32)
    h = jnp.maximum(m1t + xr + b1_ref[...], 0.0)
    y2t_ref[...] = lax.dot_general(wl2_ref[...], h, (((0,), (0,)), ((), ())),
                                   preferred_element_type=jnp.float32)
    hr2t_ref[...] = lax.dot_general(wr2_ref[...], h, (((0,), (0,)), ((), ())),
                                    preferred_element_type=jnp.float32) + b2_ref[...]


def _tc_dec_body(s2_ref, cnt_ref, hr2_ref, dw1_ref, db1_ref, dw2t_ref,
                 db2_ref, out_ref):
    c = jnp.sum(cnt_ref[...], axis=0, keepdims=True)
    cmax = jnp.maximum(c, 1.0)
    h2 = s2_ref[...] / cmax + hr2_ref[...]
    z = jnp.maximum(
        lax.dot_general(dw1_ref[...], h2, (((0,), (0,)), ((), ())),
                        preferred_element_type=jnp.float32) + db1_ref[...], 0.0)
    out_ref[...] = lax.dot_general(dw2t_ref[...], z, (((1,), (0,)), ((), ())),
                                   preferred_element_type=jnp.float32) + db2_ref[...]


_tc_pre = pl.pallas_call(
    _tc_pre_body,
    out_shape=jax.ShapeDtypeStruct((H, N), jnp.float32),
)

_tc_mid = pl.pallas_call(
    _tc_mid_body,
    out_shape=[jax.ShapeDtypeStruct((H, N), jnp.float32),
               jax.ShapeDtypeStruct((H, N), jnp.float32)],
)

_tc_dec = pl.pallas_call(
    _tc_dec_body,
    out_shape=jax.ShapeDtypeStruct((1, N), jnp.float32),
)


def kernel(x, edge_index, W_l1, W_r1, b1, W_l2, W_r2, b2, dec_w1, dec_b1,
           dec_w2, dec_b2):
    xt = x.T
    src = edge_index[0]
    dst = edge_index[1]
    y1t = _tc_pre(xt, W_l1)
    s1t, cnt = _sc_sum_cnt(y1t.reshape(-1), src, dst)
    s1t = s1t.reshape(H, N)
    cnt = cnt.reshape(NW, N)
    y2t, hr2t = _tc_mid(s1t, cnt, xt, W_r1, W_l2, W_r2,
                        b1.reshape(H, 1), b2.reshape(H, 1))
    s2t = _sc_sum(y2t.reshape(-1), src, dst).reshape(H, N)
    out = _tc_dec(s2t, cnt, hr2t, dec_w1, dec_b1.reshape(H, 1),
                  dec_w2.T, dec_b2.reshape(1, 1))
    return out.reshape(-1)


# speedup vs baseline: 7.2885x; 2.0708x over previous
import functools

import jax
import jax.numpy as jnp
from jax import lax
from jax.experimental import pallas as pl
from jax.experimental.pallas import tpu as pltpu
from jax.experimental.pallas import tpu_sc as plsc

N = 10000
E = 320000
D = 128
H = 128

NC = 2
NS = 16
NW = NC * NS
ROWS = D // NW

CH = 3200
NCHUNK = E // CH
GRP = CH // 16
CNT_PER = E // NW
CCH = 2000
NCCH = CNT_PER // CCH


def _sc_body(with_counts, y_hbm, src_hbm, dst_hbm, s_out, *rest):
    if with_counts:
        cnt_out = rest[0]
        rest = rest[1:]
    table_v = rest[0:ROWS]
    acc_v = rest[ROWS:2 * ROWS]
    src_b = rest[2 * ROWS:2 * ROWS + 2]
    dst_b = rest[2 * ROWS + 2:2 * ROWS + 4]
    sem_s = rest[2 * ROWS + 4]
    sem_d = rest[2 * ROWS + 5]
    if with_counts:
        cnt_v = rest[2 * ROWS + 6]

    wid = lax.axis_index("s") * NC + lax.axis_index("c")

    z16f = jnp.zeros((16,), jnp.float32)
    ones16 = jnp.ones((16,), jnp.float32)

    for c in range(ROWS):
        pltpu.async_copy(y_hbm.at[pl.ds((ROWS * wid + c) * N, N)],
                         table_v[c], sem_s)

    for b in range(2):
        pltpu.async_copy(src_hbm.at[pl.ds(b * CH, CH)], src_b[b], sem_s)
        pltpu.async_copy(dst_hbm.at[pl.ds(b * CH, CH)], dst_b[b], sem_d)

    for c in range(ROWS):
        @pl.loop(0, N // 16, unroll=8)
        def _(i, c=c):
            acc_v[c][pl.ds(i * 16, 16)] = z16f

    if with_counts:
        @pl.loop(0, N // 16, unroll=8)
        def _(i):
            cnt_v[pl.ds(i * 16, 16)] = z16f

    for c in range(ROWS):
        pltpu.make_async_copy(y_hbm.at[pl.ds(0, N)], table_v[c], sem_s).wait()

    @pl.loop(0, NCHUNK, step=2)
    def _(k):
        for b in range(2):
            sv, dv = src_b[b], dst_b[b]
            pltpu.make_async_copy(src_hbm.at[pl.ds(0, CH)], sv, sem_s).wait()
            pltpu.make_async_copy(dst_hbm.at[pl.ds(0, CH)], dv, sem_d).wait()

            @plsc.parallel_loop(0, GRP, unroll=8)
            def _(g, sv=sv, dv=dv):
                s16 = sv[pl.ds(g * 16, 16)]
                d16 = dv[pl.ds(g * 16, 16)]
                vals = [plsc.load_gather(table_v[c], [s16])
                        for c in range(ROWS)]
                for c in range(ROWS):
                    plsc.addupdate_scatter(acc_v[c], [d16], vals[c])

            nxt = k + 2 + b

            @pl.when(nxt < NCHUNK)
            def _(sv=sv, dv=dv, nxt=nxt):
                pltpu.async_copy(src_hbm.at[pl.ds(nxt * CH, CH)], sv, sem_s)
                pltpu.async_copy(dst_hbm.at[pl.ds(nxt * CH, CH)], dv, sem_d)

    if with_counts:
        @pl.loop(0, NCCH)
        def _(k):
            pltpu.sync_copy(
                dst_hbm.at[pl.ds(wid * CNT_PER + k * CCH, CCH)],
                dst_b[0].at[pl.ds(0, CCH)],
            )

            @pl.loop(0, CCH // 16, unroll=8)
            def _(g):
                d16 = dst_b[0][pl.ds(g * 16, 16)]
                plsc.addupdate_scatter(cnt_v, [d16], ones16)

        pltpu.sync_copy(cnt_v, cnt_out.at[pl.ds(wid * N, N)])

    for c in range(ROWS):
        pltpu.sync_copy(acc_v[c], s_out.at[pl.ds((ROWS * wid + c) * N, N)])


def _make_sc_kernel(with_counts):
    outs = [jax.ShapeDtypeStruct((D * N,), jnp.float32)]
    scratch = (
        [pltpu.VMEM((N,), jnp.float32) for _ in range(ROWS)]
        + [pltpu.VMEM((N,), jnp.float32) for _ in range(ROWS)]
        + [pltpu.VMEM((CH,), jnp.int32) for _ in range(2)]
        + [pltpu.VMEM((CH,), jnp.int32) for _ in range(2)]
        + [pltpu.SemaphoreType.DMA, pltpu.SemaphoreType.DMA]
    )
    if with_counts:
        outs.append(jax.ShapeDtypeStruct((NW * N,), jnp.float32))
        scratch.append(pltpu.VMEM((N,), jnp.float32))
    mesh = plsc.VectorSubcoreMesh(core_axis_name="c", subcore_axis_name="s")
    return pl.kernel(
        functools.partial(_sc_body, with_counts),
        out_type=tuple(outs) if with_counts else outs[0],
        mesh=mesh,
        scratch_types=scratch,
        compiler_params=pltpu.CompilerParams(needs_layout_passes=False),
        name="sage_seg_sum" + ("_cnt" if with_counts else ""),
    )


_sc_sum_cnt = _make_sc_kernel(True)
_sc_sum = _make_sc_kernel(False)


def _tc_pre_body(xt_ref, wl1_ref, y1t_ref):
    y1t_ref[...] = lax.dot_general(
        wl1_ref[...], xt_ref[...], (((0,), (0,)), ((), ())),
        preferred_element_type=jnp.float32)


def _tc_mid_body(s1_ref, cnt_ref, xt_ref, wr1_ref, wl2_ref, wr2_ref, b1_ref,
                 b2_ref, y2t_ref, hr2t_ref):
    c = jnp.sum(cnt_ref[...], axis=0, keepdims=True)
    cmax = jnp.maximum(c, 1.0)
    m1t = s1_ref[...] / cmax
    xr = lax.dot_general(wr1_ref[...], xt_ref[...], (((0,), (0,)), ((), ())),
                         preferred_element_type=jnp.float32)
    h = jnp.maximum(m1t + xr + b1_ref[...], 0.0)
    y2t_ref[...] = lax.dot_general(wl2_ref[...], h, (((0,), (0,)), ((), ())),
                                   preferred_element_type=jnp.float32)
    hr2t_ref[...] = lax.dot_general(wr2_ref[...], h, (((0,), (0,)), ((), ())),
                                    preferred_element_type=jnp.float32) + b2_ref[...]


def _tc_dec_body(s2_ref, cnt_ref, hr2_ref, dw1_ref, db1_ref, dw2t_ref,
                 db2_ref, out_ref):
    c = jnp.sum(cnt_ref[...], axis=0, keepdims=True)
    cmax = jnp.maximum(c, 1.0)
    h2 = s2_ref[...] / cmax + hr2_ref[...]
    z = jnp.maximum(
        lax.dot_general(dw1_ref[...], h2, (((0,), (0,)), ((), ())),
                        preferred_element_type=jnp.float32) + db1_ref[...], 0.0)
    out_ref[...] = lax.dot_general(dw2t_ref[...], z, (((1,), (0,)), ((), ())),
                                   preferred_element_type=jnp.float32) + db2_ref[...]


_tc_pre = pl.pallas_call(
    _tc_pre_body,
    out_shape=jax.ShapeDtypeStruct((H, N), jnp.float32),
)

_tc_mid = pl.pallas_call(
    _tc_mid_body,
    out_shape=[jax.ShapeDtypeStruct((H, N), jnp.float32),
               jax.ShapeDtypeStruct((H, N), jnp.float32)],
)

_tc_dec = pl.pallas_call(
    _tc_dec_body,
    out_shape=jax.ShapeDtypeStruct((1, N), jnp.float32),
)


def kernel(x, edge_index, W_l1, W_r1, b1, W_l2, W_r2, b2, dec_w1, dec_b1,
           dec_w2, dec_b2):
    xt = x.T
    src = edge_index[0]
    dst = edge_index[1]
    y1t = _tc_pre(xt, W_l1)
    s1t, cnt = _sc_sum_cnt(y1t.reshape(-1), src, dst)
    s1t = s1t.reshape(H, N)
    cnt = cnt.reshape(NW, N)
    y2t, hr2t = _tc_mid(s1t, cnt, xt, W_r1, W_l2, W_r2,
                        b1.reshape(H, 1), b2.reshape(H, 1))
    s2t = _sc_sum(y2t.reshape(-1), src, dst).reshape(H, N)
    out = _tc_dec(s2t, cnt, hr2t, dec_w1, dec_b1.reshape(H, 1),
                  dec_w2.T, dec_b2.reshape(1, 1))
    return out.reshape(-1)
